# baseline (speedup 1.0000x reference)
import functools

import jax
import jax.numpy as jnp
from jax import lax
from jax.experimental import pallas as pl
from jax.experimental.pallas import tpu as pltpu

_N = 10000
_D = 128
_K = 32
_R = 200
_G = _N // _R


def _a_key(x_blk, w1t, b1, w2t, b2, m1t, m2t):
    m1 = jnp.tanh(jnp.dot(x_blk, w1t, preferred_element_type=jnp.float32) + b1)
    m2 = jnp.tanh(jnp.dot(x_blk, w2t, preferred_element_type=jnp.float32) + b2)
    s = (jnp.dot(m1, m2t, preferred_element_type=jnp.float32)
         - jnp.dot(m2, m1t, preferred_element_type=jnp.float32))
    a = jnp.maximum(jnp.tanh(s), 0.0)
    return a, lax.bitcast_convert_type(a, jnp.int32)


_ONE = 0x3F800000
_SPEC = 0x3F7FFE00


def _select(key, speculative, stair_ref):
    r = key.shape[0]
    ones_col = jnp.ones((key.shape[1], 1), jnp.float32)

    def _rowsum(ind_f32):
        return jnp.dot(ind_f32, ones_col, preferred_element_type=jnp.float32)

    def _count_ge(cand):
        return _rowsum(jnp.where(key >= cand, 1.0, 0.0))

    if speculative:
        c_one = _count_ge(jnp.int32(_ONE))
        c_spec = _count_ge(jnp.int32(_SPEC))
        bad = (c_spec < _K) & (c_one < _K)
        t0 = jnp.where(c_one >= _K, jnp.int32(_ONE), jnp.int32(_SPEC))
        nbits = 9
    else:
        bad = jnp.zeros((r, 1), jnp.bool_)
        t0 = jnp.zeros((r, 1), jnp.int32)
        nbits = 30

    v = t0
    for b in range(nbits - 1, -1, -1):
        cand = v | jnp.int32(1 << b)
        v = jnp.where(_count_ge(cand) >= _K, cand, v)

    eqf = jnp.where(key == v, 1.0, 0.0)
    n_eq = _K - _rowsum(jnp.where(key > v, 1.0, 0.0))
    col = lax.broadcasted_iota(jnp.int32, key.shape, 1)

    cum = jnp.dot(eqf, stair_ref[...], preferred_element_type=jnp.float32)
    t0 = 128 * jnp.sum((cum <= n_eq).astype(jnp.int32), axis=1, keepdims=True)

    t = t0
    for b in range(6, -1, -1):
        cand = t | jnp.int32(1 << b)
        cnt = _rowsum(jnp.where(col < cand, eqf, 0.0))
        t = jnp.where(cnt <= n_eq, cand, t)
    return v, t, bad


def _mask(key, v, t):
    col = lax.broadcasted_iota(jnp.int32, key.shape, 1)
    return (key > v) | ((key == v) & (col < t))


def _mlp_t_body(xt_ref, w1_ref, b1_ref, w2_ref, b2_ref, m1t_ref, m2t_ref):
    xt = xt_ref[...]
    m1t_ref[...] = jnp.tanh(
        jnp.dot(w1_ref[...], xt, preferred_element_type=jnp.float32) + b1_ref[...])
    m2t_ref[...] = jnp.tanh(
        jnp.dot(w2_ref[...], xt, preferred_element_type=jnp.float32) + b2_ref[...])


def _phase1_body(speculative, x_ref, w1t_ref, b1_ref, w2t_ref, b2_ref,
                 m1t_ref, m2t_ref, stair_ref, vthr_ref, tcut_ref, tot_ref,
                 nbad_ref):
    a, key = _a_key(x_ref[...], w1t_ref[...], b1_ref[...], w2t_ref[...],
                    b2_ref[...], m1t_ref[...], m2t_ref[...])
    v, t, bad = _select(key, speculative, stair_ref)
    vthr_ref[...] = v
    tcut_ref[...] = t
    blk_sum = jnp.sum(jnp.where(_mask(key, v, t), a, 0.0))
    blk_bad = jnp.sum(bad.astype(jnp.int32))

    @pl.when(pl.program_id(0) == 0)
    def _():
        tot_ref[...] = jnp.zeros_like(tot_ref)
        nbad_ref[...] = jnp.zeros_like(nbad_ref)

    tot_ref[...] += blk_sum
    nbad_ref[...] += blk_bad


def _phase2_body(x_ref, w1t_ref, b1_ref, w2t_ref, b2_ref, m1t_ref, m2t_ref,
                 vthr_ref, tcut_ref, inv_ref, out_ref):
    a, key = _a_key(x_ref[...], w1t_ref[...], b1_ref[...], w2t_ref[...],
                    b2_ref[...], m1t_ref[...], m2t_ref[...])
    m = _mask(key, vthr_ref[...], tcut_ref[...])
    out_ref[...] = jnp.where(m, a * inv_ref[0, 0], 0.0)


def kernel(x, W1, b1, W2, b2, K):
    xt = x.T
    w1t = W1.T
    w2t = W2.T
    b1r = b1.reshape(1, _D)
    b2r = b2.reshape(1, _D)
    b1c = b1.reshape(_D, 1)
    b2c = b2.reshape(_D, 1)

    m1t, m2t = pl.pallas_call(
        _mlp_t_body,
        out_shape=[jax.ShapeDtypeStruct((_D, _N), jnp.float32)] * 2,
    )(xt, W1, b1c, W2, b2c)

    full = lambda shape: pl.BlockSpec(shape, lambda i: (0, 0))
    row_specs = [
        pl.BlockSpec((_R, _D), lambda i: (i, 0)),
        full((_D, _D)), full((1, _D)),
        full((_D, _D)), full((1, _D)),
        full((_D, _N)), full((_D, _N)),
    ]

    stair = (lax.broadcasted_iota(jnp.int32, (_N, 128), 0)
             < 128 * (lax.broadcasted_iota(jnp.int32, (_N, 128), 1) + 1)
             ).astype(jnp.float32)

    def _phase1(speculative, *ops):
        return pl.pallas_call(
            functools.partial(_phase1_body, speculative),
            grid=(_G,),
            in_specs=row_specs + [full((_N, 128))],
            out_specs=[
                pl.BlockSpec((_R, 1), lambda i: (i, 0)),
                pl.BlockSpec((_R, 1), lambda i: (i, 0)),
                pl.BlockSpec((1, 1), lambda i: (0, 0)),
                pl.BlockSpec((1, 1), lambda i: (0, 0)),
            ],
            out_shape=[
                jax.ShapeDtypeStruct((_N, 1), jnp.int32),
                jax.ShapeDtypeStruct((_N, 1), jnp.int32),
                jax.ShapeDtypeStruct((1, 1), jnp.float32),
                jax.ShapeDtypeStruct((1, 1), jnp.int32),
            ],
            compiler_params=pltpu.CompilerParams(
                dimension_semantics=("arbitrary",)),
        )(*ops)

    ops = (x, w1t, b1r, w2t, b2r, m1t, m2t, stair)
    vthr, tcut, tot, nbad = _phase1(True, *ops)

    vthr, tcut, tot = lax.cond(
        nbad[0, 0] > 0,
        lambda o: _phase1(False, *o)[:3],
        lambda o: (vthr, tcut, tot),
        ops)

    inv_mean = ((K * _N).astype(jnp.float32) if hasattr(K, "astype")
                else jnp.float32(K * _N)) / tot[0, 0]
    inv_arr = jnp.reshape(inv_mean, (1, 1)).astype(jnp.float32)

    out = pl.pallas_call(
        _phase2_body,
        grid=(_G,),
        in_specs=row_specs + [
            pl.BlockSpec((_R, 1), lambda i: (i, 0)),
            pl.BlockSpec((_R, 1), lambda i: (i, 0)),
            pl.BlockSpec((1, 1), lambda i: (0, 0)),
        ],
        out_specs=pl.BlockSpec((_R, _N), lambda i: (i, 0)),
        out_shape=jax.ShapeDtypeStruct((_N, _N), jnp.float32),
        compiler_params=pltpu.CompilerParams(
            dimension_semantics=("arbitrary",)),
    )(x, w1t, b1r, w2t, b2r, m1t, m2t, vthr, tcut, inv_arr)

    return out


# speedup vs baseline: 1.7757x; 1.7757x over previous
import functools

import jax
import jax.numpy as jnp
from jax import lax
from jax.experimental import pallas as pl
from jax.experimental.pallas import tpu as pltpu

_N = 10000
_D = 128
_K = 32
_R = 200
_G = _N // _R


def _a_key(x_blk, w1t, b1, w2t, b2, m1t, m2t):
    m1 = jnp.tanh(jnp.dot(x_blk, w1t, preferred_element_type=jnp.float32) + b1)
    m2 = jnp.tanh(jnp.dot(x_blk, w2t, preferred_element_type=jnp.float32) + b2)
    s = (jnp.dot(m1, m2t, preferred_element_type=jnp.float32)
         - jnp.dot(m2, m1t, preferred_element_type=jnp.float32))
    a = jnp.maximum(jnp.tanh(s), 0.0)
    return a, lax.bitcast_convert_type(a, jnp.int32)


_ONE = 0x3F800000
_SPEC = 0x3F7FFE00


def _select(key, speculative, stair_ref):
    r = key.shape[0]

    def _count_ge(cand):
        return jnp.sum((key >= cand).astype(jnp.int32), axis=1, keepdims=True)

    if speculative:
        c_one = _count_ge(jnp.int32(_ONE))
        c_spec = _count_ge(jnp.int32(_SPEC))
        bad = (c_spec < _K) & (c_one < _K)
        t0 = jnp.where(c_one >= _K, jnp.int32(_ONE), jnp.int32(_SPEC))
        nbits = 9
    else:
        bad = jnp.zeros((r, 1), jnp.bool_)
        t0 = jnp.zeros((r, 1), jnp.int32)
        nbits = 30

    v = t0
    for b in range(nbits - 1, -1, -1):
        cand = v | jnp.int32(1 << b)
        v = jnp.where(_count_ge(cand) >= _K, cand, v)

    eqf = jnp.where(key == v, 1.0, 0.0)
    eq = eqf.astype(jnp.int32)
    n_eq = _K - jnp.sum((key > v).astype(jnp.int32), axis=1, keepdims=True)
    col = lax.broadcasted_iota(jnp.int32, key.shape, 1)

    cum = jnp.dot(eqf, stair_ref[...], preferred_element_type=jnp.float32)
    n_eq_f = n_eq.astype(jnp.float32)
    t0 = 128 * jnp.sum((cum <= n_eq_f).astype(jnp.int32), axis=1, keepdims=True)

    t = t0
    for b in range(6, -1, -1):
        cand = t | jnp.int32(1 << b)
        cnt = jnp.sum(jnp.where(col < cand, eq, 0), axis=1, keepdims=True)
        t = jnp.where(cnt <= n_eq, cand, t)
    return v, t, bad


def _mask(key, v, t):
    col = lax.broadcasted_iota(jnp.int32, key.shape, 1)
    return (key > v) | ((key == v) & (col < t))


def _mlp_t_body(xt_ref, w1_ref, b1_ref, w2_ref, b2_ref, m1t_ref, m2t_ref):
    xt = xt_ref[...]
    m1t_ref[...] = jnp.tanh(
        jnp.dot(w1_ref[...], xt, preferred_element_type=jnp.float32) + b1_ref[...])
    m2t_ref[...] = jnp.tanh(
        jnp.dot(w2_ref[...], xt, preferred_element_type=jnp.float32) + b2_ref[...])


def _phase1_body(speculative, x_ref, w1t_ref, b1_ref, w2t_ref, b2_ref,
                 m1t_ref, m2t_ref, stair_ref, vthr_ref, tcut_ref, tot_ref,
                 nbad_ref):
    a, key = _a_key(x_ref[...], w1t_ref[...], b1_ref[...], w2t_ref[...],
                    b2_ref[...], m1t_ref[...], m2t_ref[...])
    v, t, bad = _select(key, speculative, stair_ref)
    vthr_ref[...] = v
    tcut_ref[...] = t
    blk_sum = jnp.sum(jnp.where(_mask(key, v, t), a, 0.0))
    blk_bad = jnp.sum(bad.astype(jnp.int32))

    @pl.when(pl.program_id(0) == 0)
    def _():
        tot_ref[...] = jnp.zeros_like(tot_ref)
        nbad_ref[...] = jnp.zeros_like(nbad_ref)

    tot_ref[...] += blk_sum
    nbad_ref[...] += blk_bad


def _phase2_body(x_ref, w1t_ref, b1_ref, w2t_ref, b2_ref, m1t_ref, m2t_ref,
                 vthr_ref, tcut_ref, inv_ref, out_ref):
    a, key = _a_key(x_ref[...], w1t_ref[...], b1_ref[...], w2t_ref[...],
                    b2_ref[...], m1t_ref[...], m2t_ref[...])
    m = _mask(key, vthr_ref[...], tcut_ref[...])
    out_ref[...] = jnp.where(m, a * inv_ref[0, 0], 0.0)


def kernel(x, W1, b1, W2, b2, K):
    xt = x.T
    w1t = W1.T
    w2t = W2.T
    b1r = b1.reshape(1, _D)
    b2r = b2.reshape(1, _D)
    b1c = b1.reshape(_D, 1)
    b2c = b2.reshape(_D, 1)

    m1t, m2t = pl.pallas_call(
        _mlp_t_body,
        out_shape=[jax.ShapeDtypeStruct((_D, _N), jnp.float32)] * 2,
    )(xt, W1, b1c, W2, b2c)

    full = lambda shape: pl.BlockSpec(shape, lambda i: (0, 0))
    row_specs = [
        pl.BlockSpec((_R, _D), lambda i: (i, 0)),
        full((_D, _D)), full((1, _D)),
        full((_D, _D)), full((1, _D)),
        full((_D, _N)), full((_D, _N)),
    ]

    stair = (lax.broadcasted_iota(jnp.int32, (_N, 128), 0)
             < 128 * (lax.broadcasted_iota(jnp.int32, (_N, 128), 1) + 1)
             ).astype(jnp.float32)

    def _phase1(speculative, *ops):
        return pl.pallas_call(
            functools.partial(_phase1_body, speculative),
            grid=(_G,),
            in_specs=row_specs + [full((_N, 128))],
            out_specs=[
                pl.BlockSpec((_R, 1), lambda i: (i, 0)),
                pl.BlockSpec((_R, 1), lambda i: (i, 0)),
                pl.BlockSpec((1, 1), lambda i: (0, 0)),
                pl.BlockSpec((1, 1), lambda i: (0, 0)),
            ],
            out_shape=[
                jax.ShapeDtypeStruct((_N, 1), jnp.int32),
                jax.ShapeDtypeStruct((_N, 1), jnp.int32),
                jax.ShapeDtypeStruct((1, 1), jnp.float32),
                jax.ShapeDtypeStruct((1, 1), jnp.int32),
            ],
            compiler_params=pltpu.CompilerParams(
                dimension_semantics=("arbitrary",)),
        )(*ops)

    ops = (x, w1t, b1r, w2t, b2r, m1t, m2t, stair)
    vthr, tcut, tot, nbad = _phase1(True, *ops)

    vthr, tcut, tot = lax.cond(
        nbad[0, 0] > 0,
        lambda o: _phase1(False, *o)[:3],
        lambda o: (vthr, tcut, tot),
        ops)

    inv_mean = ((K * _N).astype(jnp.float32) if hasattr(K, "astype")
                else jnp.float32(K * _N)) / tot[0, 0]
    inv_arr = jnp.reshape(inv_mean, (1, 1)).astype(jnp.float32)

    out = pl.pallas_call(
        _phase2_body,
        grid=(_G,),
        in_specs=row_specs + [
            pl.BlockSpec((_R, 1), lambda i: (i, 0)),
            pl.BlockSpec((_R, 1), lambda i: (i, 0)),
            pl.BlockSpec((1, 1), lambda i: (0, 0)),
        ],
        out_specs=pl.BlockSpec((_R, _N), lambda i: (i, 0)),
        out_shape=jax.ShapeDtypeStruct((_N, _N), jnp.float32),
        compiler_params=pltpu.CompilerParams(
            dimension_semantics=("arbitrary",)),
    )(x, w1t, b1r, w2t, b2r, m1t, m2t, vthr, tcut, inv_arr)

    return out


# speedup vs baseline: 1.8003x; 1.0139x over previous
import functools

import jax
import jax.numpy as jnp
from jax import lax
from jax.experimental import pallas as pl
from jax.experimental.pallas import tpu as pltpu

_N = 10000
_D = 128
_K = 32
_R = 200
_G = _N // _R


def _a_key(x_blk, w1t, b1, w2t, b2, m1t, m2t):
    m1 = jnp.tanh(jnp.dot(x_blk, w1t, preferred_element_type=jnp.float32) + b1)
    m2 = jnp.tanh(jnp.dot(x_blk, w2t, preferred_element_type=jnp.float32) + b2)
    s = (jnp.dot(m1, m2t, preferred_element_type=jnp.float32)
         - jnp.dot(m2, m1t, preferred_element_type=jnp.float32))
    a = jnp.maximum(jnp.tanh(s), 0.0)
    return a, lax.bitcast_convert_type(a, jnp.int32)


_ONE = 0x3F800000
_SPEC = 0x3F7FFE00


def _select(key, speculative, stair_ref):
    r = key.shape[0]

    def _count_ge(cand):
        return jnp.sum((key >= cand).astype(jnp.int32), axis=1, keepdims=True)

    if speculative:
        pair = jnp.sum(
            jnp.where(key >= _ONE, 16385,
                      jnp.where(key >= _SPEC, 1, 0)).astype(jnp.int32),
            axis=1, keepdims=True)
        c_spec = pair & 16383
        c_one = pair >> 14
        bad = (c_spec < _K) & (c_one < _K)
        t0 = jnp.where(c_one >= _K, jnp.int32(_ONE), jnp.int32(_SPEC))
        nbits = 9
    else:
        bad = jnp.zeros((r, 1), jnp.bool_)
        t0 = jnp.zeros((r, 1), jnp.int32)
        nbits = 30

    v = t0
    for b in range(nbits - 1, -1, -1):
        cand = v | jnp.int32(1 << b)
        v = jnp.where(_count_ge(cand) >= _K, cand, v)

    eqf = jnp.where(key == v, 1.0, 0.0)
    eq = eqf.astype(jnp.int32)
    n_eq = _K - jnp.sum((key > v).astype(jnp.int32), axis=1, keepdims=True)
    col = lax.broadcasted_iota(jnp.int32, key.shape, 1)

    cum = jnp.dot(eqf, stair_ref[...], preferred_element_type=jnp.float32)
    n_eq_f = n_eq.astype(jnp.float32)
    t0 = 128 * jnp.sum((cum <= n_eq_f).astype(jnp.int32), axis=1, keepdims=True)

    t = t0
    for b in range(6, -1, -1):
        cand = t | jnp.int32(1 << b)
        cnt = jnp.sum(jnp.where(col < cand, eq, 0), axis=1, keepdims=True)
        t = jnp.where(cnt <= n_eq, cand, t)
    return v, t, bad


def _mask(key, v, t):
    col = lax.broadcasted_iota(jnp.int32, key.shape, 1)
    return (key > v) | ((key == v) & (col < t))


def _mlp_t_body(xt_ref, w1_ref, b1_ref, w2_ref, b2_ref, m1t_ref, m2t_ref):
    xt = xt_ref[...]
    m1t_ref[...] = jnp.tanh(
        jnp.dot(w1_ref[...], xt, preferred_element_type=jnp.float32) + b1_ref[...])
    m2t_ref[...] = jnp.tanh(
        jnp.dot(w2_ref[...], xt, preferred_element_type=jnp.float32) + b2_ref[...])


def _phase1_body(speculative, x_ref, w1t_ref, b1_ref, w2t_ref, b2_ref,
                 m1t_ref, m2t_ref, stair_ref, vthr_ref, tcut_ref, tot_ref,
                 nbad_ref):
    a, key = _a_key(x_ref[...], w1t_ref[...], b1_ref[...], w2t_ref[...],
                    b2_ref[...], m1t_ref[...], m2t_ref[...])
    v, t, bad = _select(key, speculative, stair_ref)
    vthr_ref[...] = v
    tcut_ref[...] = t
    blk_sum = jnp.sum(jnp.where(_mask(key, v, t), a, 0.0))
    blk_bad = jnp.sum(bad.astype(jnp.int32))

    @pl.when(pl.program_id(0) == 0)
    def _():
        tot_ref[...] = jnp.zeros_like(tot_ref)
        nbad_ref[...] = jnp.zeros_like(nbad_ref)

    tot_ref[...] += blk_sum
    nbad_ref[...] += blk_bad


def _phase2_body(x_ref, w1t_ref, b1_ref, w2t_ref, b2_ref, m1t_ref, m2t_ref,
                 vthr_ref, tcut_ref, inv_ref, out_ref):
    a, key = _a_key(x_ref[...], w1t_ref[...], b1_ref[...], w2t_ref[...],
                    b2_ref[...], m1t_ref[...], m2t_ref[...])
    m = _mask(key, vthr_ref[...], tcut_ref[...])
    out_ref[...] = jnp.where(m, a * inv_ref[0, 0], 0.0)


def kernel(x, W1, b1, W2, b2, K):
    xt = x.T
    w1t = W1.T
    w2t = W2.T
    b1r = b1.reshape(1, _D)
    b2r = b2.reshape(1, _D)
    b1c = b1.reshape(_D, 1)
    b2c = b2.reshape(_D, 1)

    m1t, m2t = pl.pallas_call(
        _mlp_t_body,
        out_shape=[jax.ShapeDtypeStruct((_D, _N), jnp.float32)] * 2,
    )(xt, W1, b1c, W2, b2c)

    full = lambda shape: pl.BlockSpec(shape, lambda i: (0, 0))
    row_specs = [
        pl.BlockSpec((_R, _D), lambda i: (i, 0)),
        full((_D, _D)), full((1, _D)),
        full((_D, _D)), full((1, _D)),
        full((_D, _N)), full((_D, _N)),
    ]

    stair = (lax.broadcasted_iota(jnp.int32, (_N, 128), 0)
             < 128 * (lax.broadcasted_iota(jnp.int32, (_N, 128), 1) + 1)
             ).astype(jnp.float32)

    def _phase1(speculative, *ops):
        return pl.pallas_call(
            functools.partial(_phase1_body, speculative),
            grid=(_G,),
            in_specs=row_specs + [full((_N, 128))],
            out_specs=[
                pl.BlockSpec((_R, 1), lambda i: (i, 0)),
                pl.BlockSpec((_R, 1), lambda i: (i, 0)),
                pl.BlockSpec((1, 1), lambda i: (0, 0)),
                pl.BlockSpec((1, 1), lambda i: (0, 0)),
            ],
            out_shape=[
                jax.ShapeDtypeStruct((_N, 1), jnp.int32),
                jax.ShapeDtypeStruct((_N, 1), jnp.int32),
                jax.ShapeDtypeStruct((1, 1), jnp.float32),
                jax.ShapeDtypeStruct((1, 1), jnp.int32),
            ],
            compiler_params=pltpu.CompilerParams(
                dimension_semantics=("arbitrary",)),
        )(*ops)

    ops = (x, w1t, b1r, w2t, b2r, m1t, m2t, stair)
    vthr, tcut, tot, nbad = _phase1(True, *ops)

    vthr, tcut, tot = lax.cond(
        nbad[0, 0] > 0,
        lambda o: _phase1(False, *o)[:3],
        lambda o: (vthr, tcut, tot),
        ops)

    inv_mean = ((K * _N).astype(jnp.float32) if hasattr(K, "astype")
                else jnp.float32(K * _N)) / tot[0, 0]
    inv_arr = jnp.reshape(inv_mean, (1, 1)).astype(jnp.float32)

    out = pl.pallas_call(
        _phase2_body,
        grid=(_G,),
        in_specs=row_specs + [
            pl.BlockSpec((_R, 1), lambda i: (i, 0)),
            pl.BlockSpec((_R, 1), lambda i: (i, 0)),
            pl.BlockSpec((1, 1), lambda i: (0, 0)),
        ],
        out_specs=pl.BlockSpec((_R, _N), lambda i: (i, 0)),
        out_shape=jax.ShapeDtypeStruct((_N, _N), jnp.float32),
        compiler_params=pltpu.CompilerParams(
            dimension_semantics=("parallel",)),
    )(x, w1t, b1r, w2t, b2r, m1t, m2t, vthr, tcut, inv_arr)

    return out


# speedup vs baseline: 1.8014x; 1.0006x over previous
import functools

import jax
import jax.numpy as jnp
from jax import lax
from jax.experimental import pallas as pl
from jax.experimental.pallas import tpu as pltpu

_N = 10000
_D = 128
_K = 32
_R = 200
_G = _N // _R


def _a_key(x_blk, w1t, b1, w2t, b2, m1t, m2t):
    m1 = jnp.tanh(jnp.dot(x_blk, w1t, preferred_element_type=jnp.float32) + b1)
    m2 = jnp.tanh(jnp.dot(x_blk, w2t, preferred_element_type=jnp.float32) + b2)
    s = (jnp.dot(m1, m2t, preferred_element_type=jnp.float32)
         - jnp.dot(m2, m1t, preferred_element_type=jnp.float32))
    a = jnp.maximum(jnp.tanh(s), 0.0)
    return a, lax.bitcast_convert_type(a, jnp.int32)


_ONE = 0x3F800000
_SPEC = 0x3F7FFE00


def _select(key, speculative, stair_ref):
    r = key.shape[0]

    def _count_ge(cand):
        return jnp.sum((key >= cand).astype(jnp.int32), axis=1, keepdims=True)

    if speculative:
        pair = jnp.sum(
            jnp.where(key >= _ONE, 16385,
                      jnp.where(key >= _SPEC, 1, 0)).astype(jnp.int32),
            axis=1, keepdims=True)
        c_spec = pair & 16383
        c_one = pair >> 14
        bad = (c_spec < _K) & (c_one < _K)
        t0 = jnp.where(c_one >= _K, jnp.int32(_ONE), jnp.int32(_SPEC))
        v = t0
        for b in range(8, -1, -1):
            cand = v | jnp.int32(1 << b)
            v = jnp.where(_count_ge(cand) >= _K, cand, v)
    else:
        bad = jnp.zeros((r, 1), jnp.bool_)
        v = jnp.zeros((r, 1), jnp.int32)
        for b in range(29, -1, -1):
            cand = v | jnp.int32(1 << b)
            v = jnp.where(_count_ge(cand) >= _K, cand, v)

    eqf = jnp.where(key == v, 1.0, 0.0)
    n_eq = _K - jnp.sum((key > v).astype(jnp.int32), axis=1, keepdims=True)
    col = lax.broadcasted_iota(jnp.int32, key.shape, 1)

    cum = jnp.dot(eqf, stair_ref[...], preferred_element_type=jnp.float32)
    n_eq_f = n_eq.astype(jnp.float32)
    t0 = 128 * jnp.sum((cum <= n_eq_f).astype(jnp.int32), axis=1, keepdims=True)

    eq = eqf.astype(jnp.int32)
    t = t0
    for b in range(6, -1, -1):
        cand = t | jnp.int32(1 << b)
        cnt = jnp.sum(jnp.where(col < cand, eq, 0), axis=1, keepdims=True)
        t = jnp.where(cnt <= n_eq, cand, t)
    return v, t, bad


def _mask(key, v, t):
    col = lax.broadcasted_iota(jnp.int32, key.shape, 1)
    return (key > v) | ((key == v) & (col < t))


def _mlp_t_body(xt_ref, w1_ref, b1_ref, w2_ref, b2_ref, m1t_ref, m2t_ref):
    xt = xt_ref[...]
    m1t_ref[...] = jnp.tanh(
        jnp.dot(w1_ref[...], xt, preferred_element_type=jnp.float32) + b1_ref[...])
    m2t_ref[...] = jnp.tanh(
        jnp.dot(w2_ref[...], xt, preferred_element_type=jnp.float32) + b2_ref[...])


def _phase1_body(speculative, x_ref, w1t_ref, b1_ref, w2t_ref, b2_ref,
                 m1t_ref, m2t_ref, stair_ref, vthr_ref, tcut_ref, tot_ref,
                 nbad_ref):
    a, key = _a_key(x_ref[...], w1t_ref[...], b1_ref[...], w2t_ref[...],
                    b2_ref[...], m1t_ref[...], m2t_ref[...])
    v, t, bad = _select(key, speculative, stair_ref)
    vthr_ref[...] = v
    tcut_ref[...] = t
    blk_sum = jnp.sum(jnp.where(_mask(key, v, t), a, 0.0))
    blk_bad = jnp.sum(bad.astype(jnp.int32))

    @pl.when(pl.program_id(0) == 0)
    def _():
        tot_ref[...] = jnp.zeros_like(tot_ref)
        nbad_ref[...] = jnp.zeros_like(nbad_ref)

    tot_ref[...] += blk_sum
    nbad_ref[...] += blk_bad


def _phase2_body(x_ref, w1t_ref, b1_ref, w2t_ref, b2_ref, m1t_ref, m2t_ref,
                 vthr_ref, tcut_ref, inv_ref, out_ref):
    a, key = _a_key(x_ref[...], w1t_ref[...], b1_ref[...], w2t_ref[...],
                    b2_ref[...], m1t_ref[...], m2t_ref[...])
    m = _mask(key, vthr_ref[...], tcut_ref[...])
    out_ref[...] = jnp.where(m, a * inv_ref[0, 0], 0.0)


def kernel(x, W1, b1, W2, b2, K):
    xt = x.T
    w1t = W1.T
    w2t = W2.T
    b1r = b1.reshape(1, _D)
    b2r = b2.reshape(1, _D)
    b1c = b1.reshape(_D, 1)
    b2c = b2.reshape(_D, 1)

    m1t, m2t = pl.pallas_call(
        _mlp_t_body,
        out_shape=[jax.ShapeDtypeStruct((_D, _N), jnp.float32)] * 2,
    )(xt, W1, b1c, W2, b2c)

    full = lambda shape: pl.BlockSpec(shape, lambda i: (0, 0))
    row_specs = [
        pl.BlockSpec((_R, _D), lambda i: (i, 0)),
        full((_D, _D)), full((1, _D)),
        full((_D, _D)), full((1, _D)),
        full((_D, _N)), full((_D, _N)),
    ]

    stair = (lax.broadcasted_iota(jnp.int32, (_N, 128), 0)
             < 128 * (lax.broadcasted_iota(jnp.int32, (_N, 128), 1) + 1)
             ).astype(jnp.float32)

    def _phase1(speculative, *ops):
        return pl.pallas_call(
            functools.partial(_phase1_body, speculative),
            grid=(_G,),
            in_specs=row_specs + [full((_N, 128))],
            out_specs=[
                pl.BlockSpec((_R, 1), lambda i: (i, 0)),
                pl.BlockSpec((_R, 1), lambda i: (i, 0)),
                pl.BlockSpec((1, 1), lambda i: (0, 0)),
                pl.BlockSpec((1, 1), lambda i: (0, 0)),
            ],
            out_shape=[
                jax.ShapeDtypeStruct((_N, 1), jnp.int32),
                jax.ShapeDtypeStruct((_N, 1), jnp.int32),
                jax.ShapeDtypeStruct((1, 1), jnp.float32),
                jax.ShapeDtypeStruct((1, 1), jnp.int32),
            ],
            compiler_params=pltpu.CompilerParams(
                dimension_semantics=("arbitrary",)),
        )(*ops)

    ops = (x, w1t, b1r, w2t, b2r, m1t, m2t, stair)
    vthr, tcut, tot, nbad = _phase1(True, *ops)

    vthr, tcut, tot = lax.cond(
        nbad[0, 0] > 0,
        lambda o: _phase1(False, *o)[:3],
        lambda o: (vthr, tcut, tot),
        ops)

    inv_mean = ((K * _N).astype(jnp.float32) if hasattr(K, "astype")
                else jnp.float32(K * _N)) / tot[0, 0]
    inv_arr = jnp.reshape(inv_mean, (1, 1)).astype(jnp.float32)

    out = pl.pallas_call(
        _phase2_body,
        grid=(_G,),
        in_specs=row_specs + [
            pl.BlockSpec((_R, 1), lambda i: (i, 0)),
            pl.BlockSpec((_R, 1), lambda i: (i, 0)),
            pl.BlockSpec((1, 1), lambda i: (0, 0)),
        ],
        out_specs=pl.BlockSpec((_R, _N), lambda i: (i, 0)),
        out_shape=jax.ShapeDtypeStruct((_N, _N), jnp.float32),
        compiler_params=pltpu.CompilerParams(
            dimension_semantics=("parallel",)),
    )(x, w1t, b1r, w2t, b2r, m1t, m2t, vthr, tcut, inv_arr)

    return out
